# baseline (speedup 1.0000x reference)
import jax
import jax.numpy as jnp
from jax import lax
from jax.experimental import pallas as pl
from jax.experimental.pallas import tpu as pltpu

_BN_EPS = 1e-5
_NORM_EPS = 1e-12
_DIFFPOOL_EPS = 1e-15
_VMEM_LIMIT = 48 * 1024 * 1024


def _inv_deg(adj):
    return 1.0 / jnp.maximum(jnp.sum(adj, axis=-1, keepdims=True), 1.0)


def _l2norm(out):
    ss = jnp.sum(out * out, axis=-1, keepdims=True)
    return out * lax.rsqrt(jnp.maximum(ss, _NORM_EPS * _NORM_EPS))


def _sage(cat, w_ref, b):
    out = jnp.dot(cat, w_ref[...], preferred_element_type=jnp.float32) + b
    return _l2norm(out)


def _bn_stats(h, inv_bn):
    mean = jnp.sum(h, axis=0, keepdims=True) * inv_bn
    ex2 = jnp.sum(h * h, axis=0, keepdims=True) * inv_bn
    var = jnp.maximum(ex2 - mean * mean, 0.0)
    return mean, lax.rsqrt(var + _BN_EPS)


def _mono_body(x_ref, adj_ref, gd_ref, w1pe_ref, w2p_ref, w2e_ref,
               w3p_ref, w3e_ref, wlin_ref, vec_ref,
               out_x_ref, out_adj_ref, s_ref, link_ref, ent_ref,
               adjs, r1ps, r1es, r2ps, r2es, sts, a2s):
    p = pl.program_id(0)
    b = pl.program_id(1)
    vec = vec_ref[...]
    B, N, _ = adjs.shape
    H = r1ps.shape[2]
    inv_bn = 1.0 / float(B * N)

    @pl.when(p == 0)
    def _phase0():
        adj = adj_ref[...]
        adjs[b] = adj
        x = x_ref[...]
        sum_adj2 = jnp.sum(jnp.sum(adj * adj, axis=1, keepdims=True),
                           axis=0, keepdims=True)
        a2s[b] = sum_adj2 * jnp.ones((8, 128), jnp.float32)

        agg = (jnp.dot(adj, x, preferred_element_type=jnp.float32)
               * _inv_deg(adj))
        cat = jnp.concatenate([agg, x], axis=-1)
        z1 = jnp.dot(cat, w1pe_ref[...], preferred_element_type=jnp.float32)
        r1p = jnp.maximum(_l2norm(z1[:, :H] + vec[0:1]), 0.0)
        r1e = jnp.maximum(_l2norm(z1[:, H:] + vec[1:2]), 0.0)
        r1ps[b] = r1p
        r1es[b] = r1e

    @pl.when(p == 1)
    def _phase1():
        @pl.when(b == 0)
        def _stats1():
            NR = B * N
            mp_, rsp_ = _bn_stats(r1ps[...].reshape(NR, H), inv_bn)
            me_, rse_ = _bn_stats(r1es[...].reshape(NR, H), inv_bn)
            sts[0:4, :] = jnp.concatenate([mp_, rsp_, me_, rse_], axis=0)

        adj = adjs[b]
        h1p = (r1ps[b] - sts[0:1, :]) * sts[1:2, :] * vec[2:3] + vec[3:4]
        h1e = (r1es[b] - sts[2:3, :]) * sts[3:4, :] * vec[4:5] + vec[5:6]
        agg = (jnp.dot(adj, jnp.concatenate([h1p, h1e], axis=-1),
                       preferred_element_type=jnp.float32) * _inv_deg(adj))
        r2p = jnp.maximum(
            _sage(jnp.concatenate([agg[:, :H], h1p], axis=-1), w2p_ref,
                  vec[6:7]), 0.0)
        r2e = jnp.maximum(
            _sage(jnp.concatenate([agg[:, H:], h1e], axis=-1), w2e_ref,
                  vec[7:8]), 0.0)
        r2ps[b] = r2p
        r2es[b] = r2e
        r1ps[b] = h1p
        r1es[b] = h1e

    @pl.when(p == 2)
    def _phase2():
        @pl.when(b == 0)
        def _stats2():
            NR = B * N
            mp_, rsp_ = _bn_stats(r2ps[...].reshape(NR, H), inv_bn)
            me_, rse_ = _bn_stats(r2es[...].reshape(NR, H), inv_bn)
            sts[8:12, :] = jnp.concatenate([mp_, rsp_, me_, rse_], axis=0)

        adj = adjs[b]
        h1p = r1ps[b]
        h1e = r1es[b]
        h2p = (r2ps[b] - sts[8:9, :]) * sts[9:10, :] * vec[8:9] + vec[9:10]
        h2e = (r2es[b] - sts[10:11, :]) * sts[11:12, :] * vec[10:11] \
            + vec[11:12]

        agg = (jnp.dot(adj, jnp.concatenate([h2p, h2e], axis=-1),
                       preferred_element_type=jnp.float32) * _inv_deg(adj))
        h3p = _sage(jnp.concatenate([agg[:, :H], h2p], axis=-1), w3p_ref,
                    vec[12:13])
        h3e = _sage(jnp.concatenate([agg[:, H:], h2e], axis=-1), w3e_ref,
                    vec[13:14])

        logits = (jnp.dot(jnp.concatenate([h1p, h2p, h3p], axis=-1),
                          wlin_ref[...], preferred_element_type=jnp.float32)
                  + vec[14:15])
        m = jnp.max(logits, axis=-1, keepdims=True)
        e = jnp.exp(logits - m)
        sb = e / jnp.sum(e, axis=-1, keepdims=True)
        s_ref[...] = sb

        xb = jnp.concatenate([h1e, h2e, h3e], axis=-1)
        cT = (((0,), (0,)), ((), ()))
        out_x_ref[...] = lax.dot_general(sb, xb, cT,
                                         preferred_element_type=jnp.float32)
        sta = lax.dot_general(sb, adj, cT, preferred_element_type=jnp.float32)
        pooled = jnp.dot(sta, sb, preferred_element_type=jnp.float32)
        sts_mat = lax.dot_general(sb, sb, cT,
                                  preferred_element_type=jnp.float32)

        K = sb.shape[1]
        row = lax.broadcasted_iota(jnp.int32, (K, K), 0)
        col = lax.broadcasted_iota(jnp.int32, (K, K), 1)
        diag = row == col

        sum_adj2 = a2s[b][0:1, 0:1]
        tr_pooled = jnp.sum(jnp.sum(jnp.where(diag, pooled, 0.0),
                                    axis=1, keepdims=True),
                            axis=0, keepdims=True)
        sum_sts2 = jnp.sum(jnp.sum(sts_mat * sts_mat, axis=1, keepdims=True),
                           axis=0, keepdims=True)
        link_ref[...] = sum_adj2 - 2.0 * tr_pooled + sum_sts2

        ent = -sb * jnp.log(sb + _DIFFPOOL_EPS)
        ent_ref[...] = jnp.sum(jnp.sum(ent, axis=1, keepdims=True),
                               axis=0, keepdims=True)

        mn = jnp.min(jnp.min(pooled, axis=1, keepdims=True),
                     axis=0, keepdims=True)
        mx = jnp.max(jnp.max(pooled, axis=1, keepdims=True),
                     axis=0, keepdims=True)
        an = (pooled - mn) / jnp.maximum(mx - mn, 1e-12)
        hard = jnp.where(an + gd_ref[...] >= 1.0 - an, 1.0, 0.0)
        ut = jnp.where(col >= row, hard, 0.0)
        sym = ut + ut.T
        out_adj_ref[...] = jnp.where(diag, 1.0, sym)


def kernel(x, adj, rng, pool_w_rel1, pool_b1, pool_w_root1, pool_w_rel2,
           pool_b2, pool_w_root2, pool_w_rel3, pool_b3, pool_w_root3,
           pool_bn1_w, pool_bn1_b, pool_bn2_w, pool_bn2_b, pool_w_lin,
           pool_b_lin, emb_w_rel1, emb_b1, emb_w_root1, emb_w_rel2, emb_b2,
           emb_w_root2, emb_w_rel3, emb_b3, emb_w_root3, emb_bn1_w,
           emb_bn1_b, emb_bn2_w, emb_bn2_b):
    B, N, C = x.shape
    H = pool_w_rel1.shape[1]
    K = pool_w_lin.shape[1]
    Fe = emb_w_rel3.shape[1]
    D = 2 * H + Fe

    key = jax.random.wrap_key_data(rng)
    g = jax.random.gumbel(key, (2, B, K, K), jnp.float32)
    gd = g[0] - g[1]

    def wcat(wr, wo):
        return jnp.concatenate([wr, wo], axis=0)

    w1p = wcat(pool_w_rel1, pool_w_root1)
    w2p = wcat(pool_w_rel2, pool_w_root2)
    w3p = wcat(pool_w_rel3, pool_w_root3)
    w1e = wcat(emb_w_rel1, emb_w_root1)
    w2e = wcat(emb_w_rel2, emb_w_root2)
    w3e = wcat(emb_w_rel3, emb_w_root3)
    w1pe = jnp.concatenate([w1p, w1e], axis=1)

    zrow = jnp.zeros((1, H), jnp.float32)
    vec = jnp.concatenate([pool_b1, emb_b1,
                           pool_bn1_w, pool_bn1_b, emb_bn1_w, emb_bn1_b,
                           pool_b2, emb_b2,
                           pool_bn2_w, pool_bn2_b, emb_bn2_w, emb_bn2_b,
                           pool_b3, emb_b3, pool_b_lin, zrow], axis=0)

    def _in0(shape):
        return pl.BlockSpec(shape, lambda p, b: (0,) * len(shape))

    def _phase_blk(phase, park, *shape):
        if phase == 0:
            def imap(p, b):
                return (jnp.where(p == 0, b, park),) + (0,) * len(shape)
        else:
            def imap(p, b):
                return (jnp.where(p == 2, b, 0),) + (0,) * len(shape)
        return pl.BlockSpec((None,) + shape, imap)

    out_x, new_adj, s_soft, link_p, ent_p = pl.pallas_call(
        _mono_body,
        grid=(3, B),
        in_specs=[_phase_blk(0, B - 1, N, C), _phase_blk(0, B - 1, N, N),
                  _phase_blk(2, 0, K, K), _in0((2 * C, 2 * H)),
                  _in0((2 * H, H)), _in0((2 * H, H)),
                  _in0((2 * H, H)), _in0((2 * H, H)),
                  _in0((2 * H + K, K)), _in0((16, H))],
        out_specs=(_phase_blk(2, 0, K, D), _phase_blk(2, 0, K, K),
                   _phase_blk(2, 0, N, K), _phase_blk(2, 0, 1, 1),
                   _phase_blk(2, 0, 1, 1)),
        out_shape=(jax.ShapeDtypeStruct((B, K, D), jnp.float32),
                   jax.ShapeDtypeStruct((B, K, K), jnp.float32),
                   jax.ShapeDtypeStruct((B, N, K), jnp.float32),
                   jax.ShapeDtypeStruct((B, 1, 1), jnp.float32),
                   jax.ShapeDtypeStruct((B, 1, 1), jnp.float32)),
        scratch_shapes=[pltpu.VMEM((B, N, N), jnp.float32),
                        pltpu.VMEM((B, N, H), jnp.float32),
                        pltpu.VMEM((B, N, H), jnp.float32),
                        pltpu.VMEM((B, N, H), jnp.float32),
                        pltpu.VMEM((B, N, H), jnp.float32),
                        pltpu.VMEM((16, H), jnp.float32),
                        pltpu.VMEM((B, 8, 128), jnp.float32)],
        compiler_params=pltpu.CompilerParams(
            dimension_semantics=("arbitrary", "arbitrary"),
            vmem_limit_bytes=_VMEM_LIMIT),
    )(x, adj, gd, w1pe, w2p, w2e, w3p, w3e, pool_w_lin, vec)

    link = jnp.sqrt(jnp.maximum(jnp.sum(link_p), 0.0)) / float(B * N * N)
    ent = jnp.sum(ent_p) / float(B * N)
    return out_x, new_adj, link, ent, s_soft


# speedup vs baseline: 1.0926x; 1.0926x over previous
import jax
import jax.numpy as jnp
from jax import lax
from jax.experimental import pallas as pl
from jax.experimental.pallas import tpu as pltpu

_BN_EPS = 1e-5
_NORM_EPS = 1e-12
_DIFFPOOL_EPS = 1e-15
_VMEM_LIMIT = 48 * 1024 * 1024


def _inv_deg(adj):
    return 1.0 / jnp.maximum(jnp.sum(adj, axis=-1, keepdims=True), 1.0)


def _l2norm(out):
    ss = jnp.sum(out * out, axis=-1, keepdims=True)
    return out * lax.rsqrt(jnp.maximum(ss, _NORM_EPS * _NORM_EPS))


def _sage(cat, w_ref, b):
    out = jnp.dot(cat, w_ref[...], preferred_element_type=jnp.float32) + b
    return _l2norm(out)


def _bn_stats(h, inv_bn):
    mean = jnp.sum(h, axis=0, keepdims=True) * inv_bn
    ex2 = jnp.sum(h * h, axis=0, keepdims=True) * inv_bn
    var = jnp.maximum(ex2 - mean * mean, 0.0)
    return mean, lax.rsqrt(var + _BN_EPS)


def _mono_body(x_ref, adj_ref, gd_ref, w1pe_ref, w2p_ref, w2e_ref,
               w3p_ref, w3e_ref, wlin_ref, vec_ref,
               out_x_ref, out_adj_ref, s_ref, link_ref, ent_ref,
               adjs, r1ps, r1es, r2ps, r2es, sts, a2s):
    p = pl.program_id(0)
    b = pl.program_id(1)
    vec = vec_ref[...]
    B, N, _ = adjs.shape
    H = r1ps.shape[2]
    inv_bn = 1.0 / float(B * N)

    @pl.when(p == 0)
    def _phase0():
        adj = adj_ref[...]
        adjs[b] = adj
        x = x_ref[...]
        sum_adj2 = jnp.sum(jnp.sum(adj * adj, axis=1, keepdims=True),
                           axis=0, keepdims=True)
        a2s[b] = sum_adj2 * jnp.ones((8, 128), jnp.float32)

        agg = (jnp.dot(adj, x, preferred_element_type=jnp.float32)
               * _inv_deg(adj))
        cat = jnp.concatenate([agg, x], axis=-1)
        z1 = jnp.dot(cat, w1pe_ref[...], preferred_element_type=jnp.float32)
        r1p = jnp.maximum(_l2norm(z1[:, :H] + vec[0:1]), 0.0)
        r1e = jnp.maximum(_l2norm(z1[:, H:] + vec[1:2]), 0.0)
        r1ps[b] = r1p
        r1es[b] = r1e

    @pl.when(p == 1)
    def _phase1():
        @pl.when(b == 0)
        def _stats1():
            NR = B * N
            mp_, rsp_ = _bn_stats(r1ps[...].reshape(NR, H), inv_bn)
            me_, rse_ = _bn_stats(r1es[...].reshape(NR, H), inv_bn)
            sts[0:4, :] = jnp.concatenate([mp_, rsp_, me_, rse_], axis=0)

        adj = adjs[b]
        h1p = (r1ps[b] - sts[0:1, :]) * sts[1:2, :] * vec[2:3] + vec[3:4]
        h1e = (r1es[b] - sts[2:3, :]) * sts[3:4, :] * vec[4:5] + vec[5:6]
        agg = (jnp.dot(adj, jnp.concatenate([h1p, h1e], axis=-1),
                       preferred_element_type=jnp.float32) * _inv_deg(adj))
        r2p = jnp.maximum(
            _sage(jnp.concatenate([agg[:, :H], h1p], axis=-1), w2p_ref,
                  vec[6:7]), 0.0)
        r2e = jnp.maximum(
            _sage(jnp.concatenate([agg[:, H:], h1e], axis=-1), w2e_ref,
                  vec[7:8]), 0.0)
        r2ps[b] = r2p
        r2es[b] = r2e
        r1ps[b] = h1p
        r1es[b] = h1e

    @pl.when(p == 2)
    def _phase2():
        @pl.when(b == 0)
        def _stats2():
            NR = B * N
            mp_, rsp_ = _bn_stats(r2ps[...].reshape(NR, H), inv_bn)
            me_, rse_ = _bn_stats(r2es[...].reshape(NR, H), inv_bn)
            sts[8:12, :] = jnp.concatenate([mp_, rsp_, me_, rse_], axis=0)

        adj = adjs[b]
        h1p = r1ps[b]
        h1e = r1es[b]
        h2p = (r2ps[b] - sts[8:9, :]) * sts[9:10, :] * vec[8:9] + vec[9:10]
        h2e = (r2es[b] - sts[10:11, :]) * sts[11:12, :] * vec[10:11] \
            + vec[11:12]

        agg = (jnp.dot(adj, jnp.concatenate([h2p, h2e], axis=-1),
                       preferred_element_type=jnp.float32) * _inv_deg(adj))
        h3p = _sage(jnp.concatenate([agg[:, :H], h2p], axis=-1), w3p_ref,
                    vec[12:13])
        h3e = _sage(jnp.concatenate([agg[:, H:], h2e], axis=-1), w3e_ref,
                    vec[13:14])

        logits = (jnp.dot(jnp.concatenate([h1p, h2p, h3p], axis=-1),
                          wlin_ref[...], preferred_element_type=jnp.float32)
                  + vec[14:15])
        m = jnp.max(logits, axis=-1, keepdims=True)
        e = jnp.exp(logits - m)
        sb = e / jnp.sum(e, axis=-1, keepdims=True)
        s_ref[...] = sb

        xb = jnp.concatenate([h1e, h2e, h3e], axis=-1)
        cT = (((0,), (0,)), ((), ()))
        out_x_ref[...] = lax.dot_general(sb, xb, cT,
                                         preferred_element_type=jnp.float32)
        sta = lax.dot_general(sb, adj, cT, preferred_element_type=jnp.float32)
        pooled = jnp.dot(sta, sb, preferred_element_type=jnp.float32)
        sts_mat = lax.dot_general(sb, sb, cT,
                                  preferred_element_type=jnp.float32)

        K = sb.shape[1]
        row = lax.broadcasted_iota(jnp.int32, (K, K), 0)
        col = lax.broadcasted_iota(jnp.int32, (K, K), 1)
        diag = row == col

        sum_adj2 = a2s[b][0:1, 0:1]
        tr_pooled = jnp.sum(jnp.sum(jnp.where(diag, pooled, 0.0),
                                    axis=1, keepdims=True),
                            axis=0, keepdims=True)
        sum_sts2 = jnp.sum(jnp.sum(sts_mat * sts_mat, axis=1, keepdims=True),
                           axis=0, keepdims=True)
        la = sum_adj2 - 2.0 * tr_pooled + sum_sts2

        ent = -sb * jnp.log(sb + _DIFFPOOL_EPS)
        ea = jnp.sum(jnp.sum(ent, axis=1, keepdims=True),
                     axis=0, keepdims=True)

        acc_l = jnp.where(b == 0, la, sts[12:13, 0:1] + la)
        acc_e = jnp.where(b == 0, ea, sts[13:14, 0:1] + ea)
        sts[12:13, 0:1] = acc_l
        sts[13:14, 0:1] = acc_e

        @pl.when(b == B - 1)
        def _finalize():
            link_ref[...] = (jnp.sqrt(jnp.maximum(acc_l, 0.0))
                             / float(B * N * N))
            ent_ref[...] = acc_e / float(B * N)

        mn = jnp.min(jnp.min(pooled, axis=1, keepdims=True),
                     axis=0, keepdims=True)
        mx = jnp.max(jnp.max(pooled, axis=1, keepdims=True),
                     axis=0, keepdims=True)
        an = (pooled - mn) / jnp.maximum(mx - mn, 1e-12)
        hard = jnp.where(an + gd_ref[...] >= 1.0 - an, 1.0, 0.0)
        ut = jnp.where(col >= row, hard, 0.0)
        sym = ut + ut.T
        out_adj_ref[...] = jnp.where(diag, 1.0, sym)


def kernel(x, adj, rng, pool_w_rel1, pool_b1, pool_w_root1, pool_w_rel2,
           pool_b2, pool_w_root2, pool_w_rel3, pool_b3, pool_w_root3,
           pool_bn1_w, pool_bn1_b, pool_bn2_w, pool_bn2_b, pool_w_lin,
           pool_b_lin, emb_w_rel1, emb_b1, emb_w_root1, emb_w_rel2, emb_b2,
           emb_w_root2, emb_w_rel3, emb_b3, emb_w_root3, emb_bn1_w,
           emb_bn1_b, emb_bn2_w, emb_bn2_b):
    B, N, C = x.shape
    H = pool_w_rel1.shape[1]
    K = pool_w_lin.shape[1]
    Fe = emb_w_rel3.shape[1]
    D = 2 * H + Fe

    key = jax.random.wrap_key_data(rng)
    g = jax.random.gumbel(key, (2, B, K, K), jnp.float32)
    gd = g[0] - g[1]

    def wcat(wr, wo):
        return jnp.concatenate([wr, wo], axis=0)

    w1p = wcat(pool_w_rel1, pool_w_root1)
    w2p = wcat(pool_w_rel2, pool_w_root2)
    w3p = wcat(pool_w_rel3, pool_w_root3)
    w1e = wcat(emb_w_rel1, emb_w_root1)
    w2e = wcat(emb_w_rel2, emb_w_root2)
    w3e = wcat(emb_w_rel3, emb_w_root3)
    w1pe = jnp.concatenate([w1p, w1e], axis=1)

    zrow = jnp.zeros((1, H), jnp.float32)
    vec = jnp.concatenate([pool_b1, emb_b1,
                           pool_bn1_w, pool_bn1_b, emb_bn1_w, emb_bn1_b,
                           pool_b2, emb_b2,
                           pool_bn2_w, pool_bn2_b, emb_bn2_w, emb_bn2_b,
                           pool_b3, emb_b3, pool_b_lin, zrow], axis=0)

    def _in0(shape):
        return pl.BlockSpec(shape, lambda p, b: (0,) * len(shape))

    def _phase_blk(phase, park, *shape):
        if phase == 0:
            def imap(p, b):
                return (jnp.where(p == 0, b, park),) + (0,) * len(shape)
        else:
            def imap(p, b):
                return (jnp.where(p == 2, b, 0),) + (0,) * len(shape)
        return pl.BlockSpec((None,) + shape, imap)

    out_x, new_adj, s_soft, link_p, ent_p = pl.pallas_call(
        _mono_body,
        grid=(3, B),
        in_specs=[_phase_blk(0, B - 1, N, C), _phase_blk(0, B - 1, N, N),
                  _phase_blk(2, 0, K, K), _in0((2 * C, 2 * H)),
                  _in0((2 * H, H)), _in0((2 * H, H)),
                  _in0((2 * H, H)), _in0((2 * H, H)),
                  _in0((2 * H + K, K)), _in0((16, H))],
        out_specs=(_phase_blk(2, 0, K, D), _phase_blk(2, 0, K, K),
                   _phase_blk(2, 0, N, K),
                   pl.BlockSpec((None, 1, 1), lambda p, b: (0, 0, 0)),
                   pl.BlockSpec((None, 1, 1), lambda p, b: (0, 0, 0))),
        out_shape=(jax.ShapeDtypeStruct((B, K, D), jnp.float32),
                   jax.ShapeDtypeStruct((B, K, K), jnp.float32),
                   jax.ShapeDtypeStruct((B, N, K), jnp.float32),
                   jax.ShapeDtypeStruct((1, 1, 1), jnp.float32),
                   jax.ShapeDtypeStruct((1, 1, 1), jnp.float32)),
        scratch_shapes=[pltpu.VMEM((B, N, N), jnp.float32),
                        pltpu.VMEM((B, N, H), jnp.float32),
                        pltpu.VMEM((B, N, H), jnp.float32),
                        pltpu.VMEM((B, N, H), jnp.float32),
                        pltpu.VMEM((B, N, H), jnp.float32),
                        pltpu.VMEM((16, H), jnp.float32),
                        pltpu.VMEM((B, 8, 128), jnp.float32)],
        compiler_params=pltpu.CompilerParams(
            dimension_semantics=("arbitrary", "arbitrary"),
            vmem_limit_bytes=_VMEM_LIMIT),
    )(x, adj, gd, w1pe, w2p, w2e, w3p, w3e, pool_w_lin, vec)

    return out_x, new_adj, link_p[0, 0, 0], ent_p[0, 0, 0], s_soft
